# baseline (speedup 1.0000x reference)
import functools

import jax
import jax.numpy as jnp
from jax import lax
from jax.experimental import pallas as pl
from jax.experimental.pallas import tpu as pltpu
from jax.experimental.pallas import tpu_sc as plsc

_D = 32
_H1 = 64
_H2 = 32


def _gather_rows(table, idx_flat):
    BF = idx_flat.shape[0]
    info = plsc.get_sparse_core_info()
    NC, NS = info.num_cores, info.num_subcores
    NW = NC * NS
    per_w = BF // NW
    C = 3328
    assert per_w % C == 0
    n_chunks = per_w // C
    mesh = plsc.VectorSubcoreMesh(core_axis_name="c", subcore_axis_name="s")

    @functools.partial(
        pl.kernel,
        out_type=jax.ShapeDtypeStruct((BF, _D), jnp.float32),
        mesh=mesh,
        scratch_types=[
            pltpu.VMEM((C,), jnp.int32),
            pltpu.VMEM((C, _D), jnp.float32),
            pltpu.SemaphoreType.DMA,
        ],
        compiler_params=pltpu.CompilerParams(use_tc_tiling_on_sc=False),
    )
    def gather_kernel(idx_hbm, table_hbm, out_hbm, idx_v, rows_v, sem):
        wid = lax.axis_index("s") * NC + lax.axis_index("c")
        for i in range(n_chunks):
            base = wid * per_w + i * C
            pltpu.sync_copy(idx_hbm.at[pl.ds(base, C)], idx_v)
            pltpu.async_copy(table_hbm.at[idx_v], rows_v, sem).wait()
            pltpu.sync_copy(rows_v, out_hbm.at[pl.ds(base, C)])

    return gather_kernel(idx_flat, table)


def _mlp_transposed(g4, W1r4, b1, W2, b2, F, B):
    Q = B // 4
    QC = 512
    assert Q % QC == 0
    nq = Q // QC

    def body(g_ref, w1_ref, b1_ref, w2_ref, b2_ref, o_ref):
        g = g_ref[0]
        h = jnp.dot(g, w1_ref[0], preferred_element_type=jnp.float32)
        h = jnp.maximum(h + b1_ref[...], 0.0)
        ot = jax.lax.dot_general(
            w2_ref[...], h, (((0,), (1,)), ((), ())),
            preferred_element_type=jnp.float32,
        )
        o_ref[0] = ot + b2_ref[...]

    return pl.pallas_call(
        body,
        grid=(F, nq, 4),
        in_specs=[
            pl.BlockSpec((1, QC, 4 * _D), lambda f, qb, r: (f, qb, 0)),
            pl.BlockSpec((1, 4 * _D, _H1), lambda f, qb, r: (r, 0, 0)),
            pl.BlockSpec((1, _H1), lambda f, qb, r: (0, 0)),
            pl.BlockSpec((_H1, _H2), lambda f, qb, r: (0, 0)),
            pl.BlockSpec((_H2, 1), lambda f, qb, r: (0, 0)),
        ],
        out_specs=pl.BlockSpec(
            (1, _H2, QC), lambda f, qb, r: (f, 0, r * nq + qb)
        ),
        out_shape=jax.ShapeDtypeStruct((F, _H2, B), jnp.float32),
    )(g4, W1r4, b1, W2, b2)


def kernel(x, table, W1, b1, W2, b2):
    B, F = x.shape
    Q = B // 4
    idx_flat = (
        x.T.reshape(F, 4, Q).transpose(0, 2, 1).reshape(-1).astype(jnp.int32)
    )
    g = _gather_rows(table, idx_flat)
    g4 = g.reshape(F, Q, 4 * _D)
    W1r4 = jnp.zeros((4, 4 * _D, _H1), dtype=W1.dtype)
    for r in range(4):
        W1r4 = W1r4.at[r, r * _D : (r + 1) * _D, :].set(W1)
    o3 = _mlp_transposed(
        g4, W1r4, b1.reshape(1, _H1), W2, b2.reshape(_H2, 1), F, B
    )
    return o3.transpose(2, 0, 1)


# speedup vs baseline: 1.8210x; 1.8210x over previous
import functools

import jax
import jax.numpy as jnp
from jax import lax
from jax.experimental import pallas as pl
from jax.experimental.pallas import tpu as pltpu
from jax.experimental.pallas import tpu_sc as plsc

_D = 32
_H1 = 64
_H2 = 32


def _gather_rows(table, idx_flat):
    BF = idx_flat.shape[0]
    info = plsc.get_sparse_core_info()
    NC, NS = info.num_cores, info.num_subcores
    NW = NC * NS
    per_w = BF // NW
    C = 3328
    assert per_w % C == 0
    n_chunks = per_w // C
    mesh = plsc.VectorSubcoreMesh(core_axis_name="c", subcore_axis_name="s")

    @functools.partial(
        pl.kernel,
        out_type=jax.ShapeDtypeStruct((BF, _D), jnp.float32),
        mesh=mesh,
        scratch_types=[
            pltpu.VMEM((C,), jnp.int32),
            pltpu.VMEM((C, _D), jnp.float32),
            pltpu.SemaphoreType.DMA,
        ],
        compiler_params=pltpu.CompilerParams(use_tc_tiling_on_sc=False),
    )
    def gather_kernel(idx_hbm, table_hbm, out_hbm, idx_v, rows_v, sem):
        wid = lax.axis_index("s") * NC + lax.axis_index("c")
        for i in range(n_chunks):
            base = wid * per_w + i * C
            pltpu.sync_copy(idx_hbm.at[pl.ds(base, C)], idx_v)
            pltpu.async_copy(table_hbm.at[idx_v], rows_v, sem).wait()
            pltpu.sync_copy(rows_v, out_hbm.at[pl.ds(base, C)])

    return gather_kernel(idx_flat, table)


def _mlp_transposed(g4, W1d, b1d, W2d, b2t, F, B):
    Q = B // 4

    def body(g_ref, w1_ref, b1_ref, w2_ref, b2_ref, o_ref):
        g = g_ref[0]
        h = jnp.dot(g, w1_ref[...], preferred_element_type=jnp.float32)
        h = jnp.maximum(h + b1_ref[...], 0.0)
        ot = jax.lax.dot_general(
            w2_ref[...], h, (((0,), (1,)), ((), ())),
            preferred_element_type=jnp.float32,
        )
        ot = ot + b2_ref[...]
        for r in range(4):
            o_ref[0, :, r * Q : (r + 1) * Q] = ot[r * _H2 : (r + 1) * _H2, :]

    return pl.pallas_call(
        body,
        grid=(F,),
        in_specs=[
            pl.BlockSpec((1, Q, 4 * _D), lambda f: (f, 0, 0)),
            pl.BlockSpec((4 * _D, 4 * _H1), lambda f: (0, 0)),
            pl.BlockSpec((1, 4 * _H1), lambda f: (0, 0)),
            pl.BlockSpec((4 * _H1, 4 * _H2), lambda f: (0, 0)),
            pl.BlockSpec((4 * _H2, 1), lambda f: (0, 0)),
        ],
        out_specs=pl.BlockSpec((1, _H2, B), lambda f: (f, 0, 0)),
        out_shape=jax.ShapeDtypeStruct((F, _H2, B), jnp.float32),
    )(g4, W1d, b1d, W2d, b2t)


def _block_diag4(W):
    a, b = W.shape
    out = jnp.zeros((4 * a, 4 * b), dtype=W.dtype)
    for r in range(4):
        out = out.at[r * a : (r + 1) * a, r * b : (r + 1) * b].set(W)
    return out


def kernel(x, table, W1, b1, W2, b2):
    B, F = x.shape
    Q = B // 4
    idx_flat = (
        x.T.reshape(F, 4, Q).transpose(0, 2, 1).reshape(-1).astype(jnp.int32)
    )
    g = _gather_rows(table, idx_flat)
    g4 = g.reshape(F, Q, 4 * _D)
    W1d = _block_diag4(W1)
    W2d = _block_diag4(W2)
    b1d = jnp.tile(b1, 4).reshape(1, 4 * _H1)
    b2t = jnp.tile(b2, 4).reshape(4 * _H2, 1)
    o3 = _mlp_transposed(g4, W1d, b1d, W2d, b2t, F, B)
    return o3.transpose(2, 0, 1)
